# baseline (speedup 1.0000x reference)
import functools

import jax
import jax.numpy as jnp
from jax import lax
from jax.experimental import pallas as pl
from jax.experimental.pallas import tpu as pltpu
from jax.experimental.pallas import tpu_sc as plsc

_S = 1024


def _pool_body(seg_ref, x_ref, w1_ref, b1_ref, w2_ref, out_ref, denom_ref,
               *, nb, s, sw):
    i = pl.program_id(0)

    @pl.when(i == 0)
    def _init():
        out_ref[...] = jnp.zeros_like(out_ref)
        denom_ref[...] = jnp.zeros_like(denom_ref)

    xb = x_ref[...]
    h = jnp.tanh(
        jnp.dot(xb, w1_ref[...], preferred_element_type=jnp.float32)
        + b1_ref[...])
    wv = jnp.dot(h, w2_ref[...], preferred_element_type=jnp.float32)
    e = jnp.exp(wv)

    seg = seg_ref[0]
    bn = seg.shape[-1]
    xw = (xb * e).astype(jnp.bfloat16)
    e16 = e.astype(jnp.bfloat16)
    smin = jnp.min(seg)
    smax = jnp.max(seg)
    for j in range(s // sw):
        lo = j * sw

        @pl.when(jnp.logical_and(smin < lo + sw, smax >= lo))
        def _win(lo=lo):
            pj = (jax.lax.broadcasted_iota(jnp.int32, (sw, bn), 0) + lo
                  == seg).astype(jnp.bfloat16)
            out_ref[lo:lo + sw, :] += jnp.dot(
                pj, xw, preferred_element_type=jnp.float32)
            dj = jnp.dot(pj, e16, preferred_element_type=jnp.float32)
            denom_ref[lo:lo + sw, :] += jnp.broadcast_to(dj, (sw, 16))

def _pooling_call(x, seg3, w1, b1r, w2, *, bn, nb, s, d, sw=256,
                  interpret=False):
    return pl.pallas_call(
        functools.partial(_pool_body, nb=nb, s=s, sw=sw),
        grid=(nb,),
        in_specs=[
            pl.BlockSpec((1, 1, bn), lambda i: (i, 0, 0)),
            pl.BlockSpec((bn, d), lambda i: (i, 0)),
            pl.BlockSpec((d, d), lambda i: (0, 0)),
            pl.BlockSpec((1, d), lambda i: (0, 0)),
            pl.BlockSpec((d, 1), lambda i: (0, 0)),
        ],
        out_specs=(pl.BlockSpec((s, d), lambda i: (0, 0)),
                   pl.BlockSpec((s, 16), lambda i: (0, 0))),
        out_shape=(jax.ShapeDtypeStruct((s, d), jnp.float32),
                   jax.ShapeDtypeStruct((s, 16), jnp.float32)),
        compiler_params=pltpu.CompilerParams(
            dimension_semantics=("arbitrary",)),
        interpret=interpret,
    )(seg3, x, w1, b1r, w2)


def _sc_norm_body(numer_hbm, denom_hbm, out_hbm, numer_v, denom_v, *, rows, d):
    w = lax.axis_index("c") * 16 + lax.axis_index("s")
    base = w * rows
    pltpu.sync_copy(numer_hbm.at[pl.ds(base * d, rows * d)], numer_v)
    pltpu.sync_copy(denom_hbm.at[pl.ds(base * 16, rows * 16)], denom_v)
    for r in range(rows):
        ivec = 1.0 / (denom_v[pl.ds(r * 16, 16)] + 1e-16)
        for cc in range(d // 16):
            o = r * d + cc * 16
            numer_v[pl.ds(o, 16)] = numer_v[pl.ds(o, 16)] * ivec
    pltpu.sync_copy(numer_v, out_hbm.at[pl.ds(base * d, rows * d)])


def _sc_normalize(numer_flat, denom_flat, *, s, d):
    nw = 32
    rows = s // nw
    kfn = pl.kernel(
        functools.partial(_sc_norm_body, rows=rows, d=d),
        mesh=plsc.VectorSubcoreMesh(core_axis_name="c", subcore_axis_name="s"),
        out_type=jax.ShapeDtypeStruct((s * d,), jnp.float32),
        scratch_types=[
            pltpu.VMEM((rows * d,), jnp.float32),
            pltpu.VMEM((rows * 16,), jnp.float32),
        ],
    )
    return kfn(numer_flat, denom_flat)


def kernel(x, batch, W1, b1, W2, b2):
    n, d = x.shape
    bn = 12800
    nb = n // bn
    seg3 = batch.astype(jnp.int32).reshape(nb, 1, bn)
    numer, denom = _pooling_call(x, seg3, W1, b1.reshape(1, d), W2,
                                 bn=bn, nb=nb, s=_S, d=d, sw=128)
    out_flat = _sc_normalize(numer.reshape(_S * d), denom.reshape(_S * 16),
                             s=_S, d=d)
    return out_flat.reshape(_S, d)
